# Initial kernel scaffold; baseline (speedup 1.0000x reference)
import jax
import jax.numpy as jnp
from jax.experimental import pallas as pl


def kernel(x, W, b, m0, m1, m2, m3, m4):
    raise NotImplementedError("write your pallas kernel here")


# speedup vs baseline: 1.7836x; 1.7836x over previous
import jax
import jax.numpy as jnp
from jax.experimental import pallas as pl

CLS = (30, 100, 300, 1000, 3000, 8000)
BATCH_BLK = 256


def _first_argmax(vals):
    m = jnp.max(vals, axis=1, keepdims=True)
    iota = jax.lax.broadcasted_iota(jnp.int32, vals.shape, 1)
    return jnp.min(jnp.where(vals == m, iota, jnp.int32(2**30)),
                   axis=1, keepdims=True)


def _head_kernel(x_ref, w_ref, b_ref, sm_ref, pred_ref):
    logits = jnp.dot(x_ref[...], w_ref[...],
                     preferred_element_type=jnp.float32) + b_ref[...]
    m = jnp.max(logits, axis=1, keepdims=True)
    e = jnp.exp(logits - m)
    s = jnp.sum(e, axis=1, keepdims=True)
    sm_ref[...] = e / s
    pred_ref[...] = _first_argmax(logits)


def _rowargmax_kernel(m_ref, out_ref):
    out_ref[...] = _first_argmax(m_ref[...])


def _row_argmax(mat, row_blk):
    rows, cols = mat.shape
    grid = rows // row_blk
    return pl.pallas_call(
        _rowargmax_kernel,
        grid=(grid,),
        in_specs=[pl.BlockSpec((row_blk, cols), lambda i: (i, 0))],
        out_specs=pl.BlockSpec((row_blk, 1), lambda i: (i, 0)),
        out_shape=jax.ShapeDtypeStruct((rows, 1), jnp.int32),
    )(mat)


def _chain_kernel(pred5_ref, p4_ref, p3_ref, p2_ref, p1_ref, p0_ref,
                  o4_ref, o3_ref, o2_ref, o1_ref, o0_ref):
    pred = pred5_ref[...]
    b = pred.shape[0]
    steps = ((p4_ref, o4_ref, CLS[5], CLS[4]),
             (p3_ref, o3_ref, CLS[4], CLS[3]),
             (p2_ref, o2_ref, CLS[3], CLS[2]),
             (p1_ref, o1_ref, CLS[2], CLS[1]),
             (p0_ref, o0_ref, CLS[1], CLS[0]))
    for t_ref, o_ref, dom, rng in steps:
        iota = jax.lax.broadcasted_iota(jnp.int32, (b, dom), 1)
        mask = pred == iota
        pred = jnp.sum(jnp.where(mask, t_ref[...], 0), axis=1, keepdims=True)
        iota2 = jax.lax.broadcasted_iota(jnp.int32, (b, rng), 1)
        o_ref[...] = (pred == iota2).astype(jnp.float32)


def kernel(x, W, b, m0, m1, m2, m3, m4):
    n = x.shape[0]
    d_in = x.shape[1]
    grid = n // BATCH_BLK

    sm, pred5 = pl.pallas_call(
        _head_kernel,
        grid=(grid,),
        in_specs=[
            pl.BlockSpec((BATCH_BLK, d_in), lambda i: (i, 0)),
            pl.BlockSpec((d_in, CLS[5]), lambda i: (0, 0)),
            pl.BlockSpec((1, CLS[5]), lambda i: (0, 0)),
        ],
        out_specs=[
            pl.BlockSpec((BATCH_BLK, CLS[5]), lambda i: (i, 0)),
            pl.BlockSpec((BATCH_BLK, 1), lambda i: (i, 0)),
        ],
        out_shape=[
            jax.ShapeDtypeStruct((n, CLS[5]), jnp.float32),
            jax.ShapeDtypeStruct((n, 1), jnp.int32),
        ],
    )(x, W, b.reshape(1, CLS[5]))

    p4 = _row_argmax(m4, 1000).reshape(1, CLS[5])
    p3 = _row_argmax(m3, 1000).reshape(1, CLS[4])
    p2 = _row_argmax(m2, 1000).reshape(1, CLS[3])
    p1 = _row_argmax(m1, 300).reshape(1, CLS[2])
    p0 = _row_argmax(m0, 100).reshape(1, CLS[1])

    o4, o3, o2, o1, o0 = pl.pallas_call(
        _chain_kernel,
        grid=(grid,),
        in_specs=[
            pl.BlockSpec((BATCH_BLK, 1), lambda i: (i, 0)),
            pl.BlockSpec((1, CLS[5]), lambda i: (0, 0)),
            pl.BlockSpec((1, CLS[4]), lambda i: (0, 0)),
            pl.BlockSpec((1, CLS[3]), lambda i: (0, 0)),
            pl.BlockSpec((1, CLS[2]), lambda i: (0, 0)),
            pl.BlockSpec((1, CLS[1]), lambda i: (0, 0)),
        ],
        out_specs=[
            pl.BlockSpec((BATCH_BLK, CLS[4]), lambda i: (i, 0)),
            pl.BlockSpec((BATCH_BLK, CLS[3]), lambda i: (i, 0)),
            pl.BlockSpec((BATCH_BLK, CLS[2]), lambda i: (i, 0)),
            pl.BlockSpec((BATCH_BLK, CLS[1]), lambda i: (i, 0)),
            pl.BlockSpec((BATCH_BLK, CLS[0]), lambda i: (i, 0)),
        ],
        out_shape=[
            jax.ShapeDtypeStruct((n, CLS[4]), jnp.float32),
            jax.ShapeDtypeStruct((n, CLS[3]), jnp.float32),
            jax.ShapeDtypeStruct((n, CLS[2]), jnp.float32),
            jax.ShapeDtypeStruct((n, CLS[1]), jnp.float32),
            jax.ShapeDtypeStruct((n, CLS[0]), jnp.float32),
        ],
    )(pred5, p4, p3, p2, p1, p0)

    return (o0, o1, o2, o3, o4, sm)
